# baseline (speedup 1.0000x reference)
import functools

import jax
import jax.numpy as jnp
import numpy as np
from jax.experimental import pallas as pl
from jax.experimental.pallas import tpu as pltpu

_R = 10
_MAXR = 2 * _R - 1
_TINY = np.float32(np.finfo(np.float32).tiny)


def _threefry_xor_bits(k0, k1, cnt):
    ks2 = k0 ^ k1 ^ np.uint32(0x1BD11BDA)
    ks = (k0, k1, ks2)
    x0 = jnp.zeros_like(cnt) + k0
    x1 = cnt + k1

    def rotl(v, d):
        return (v << np.uint32(d)) | (v >> np.uint32(32 - d))

    rots = ((13, 15, 26, 6), (17, 29, 16, 24))
    for i in range(5):
        for r in rots[i % 2]:
            x0 = x0 + x1
            x1 = rotl(x1, r)
            x1 = x1 ^ x0
        x0 = x0 + ks[(i + 1) % 3]
        x1 = x1 + ks[(i + 2) % 3] + np.uint32(i + 1)
    return x0 ^ x1


def _sampler_block(x_ref, w_ref, rad_ref, u_ref, row_ref, keys_ref, o_ref, q_ref,
                   *, rblk, dim):
    x0 = x_ref[...]
    w = w_ref[...]
    wh = w * np.float32(0.5)

    col = jax.lax.broadcasted_iota(jnp.int32, (rblk, dim), 1)
    flat = row_ref[...] * np.uint32(dim) + \
        jax.lax.broadcasted_iota(jnp.uint32, (rblk, dim), 1)

    s0 = (1.0 - 2.0 * x0) * wh
    m0 = jnp.max(s0, axis=-1, keepdims=True)
    log_zx = jnp.log(jnp.sum(jnp.exp(s0 - m0), axis=-1, keepdims=True)) + m0
    score_x = jnp.sum(x0 * w, axis=-1, keepdims=True)
    rad = rad_ref[...]
    t_max = jnp.max(rad)

    o_ref[...] = x0
    q_ref[...] = jnp.exp(-s0)

    def step(t, carry):
        bits = _threefry_xor_bits(keys_ref[t, 0], keys_ref[t, 1], flat)
        f = jax.lax.bitcast_convert_type(
            (bits >> np.uint32(9)) | np.uint32(0x3F800000), jnp.float32) - 1.0
        u = jnp.maximum(_TINY, f * (np.float32(1.0) - _TINY) + _TINY)
        q = q_ref[...]
        v = -jnp.log(u) * q
        m = jnp.min(v, axis=-1, keepdims=True)
        idx = jnp.min(jnp.where(v == m, col, np.int32(dim)), axis=-1, keepdims=True)
        mask = (col == idx) & (t < rad)
        xc = o_ref[...]
        o_ref[...] = jnp.where(mask, 1.0 - xc, xc)
        q_ref[...] = jnp.where(mask, 1.0 / q, q)
        return carry

    jax.lax.fori_loop(0, t_max, step, 0, unroll=False)

    y = o_ref[...]
    s_y = (1.0 - 2.0 * y) * wh
    my = jnp.max(s_y, axis=-1, keepdims=True)
    lse_y = jnp.log(jnp.sum(jnp.exp(s_y - my), axis=-1, keepdims=True)) + my
    score_y = jnp.sum(y * w, axis=-1, keepdims=True)
    log_tilde = -jnp.sum(w * (y - x0), axis=-1, keepdims=True)
    log_acc = jnp.minimum((score_y - score_x) + log_tilde + (log_zx - lse_y), 0.0)
    acc = jnp.exp(log_acc) >= u_ref[...]
    o_ref[...] = jnp.where(acc, y, x0)


@jax.jit
def kernel(x, W):
    bsize, dim = x.shape
    key = jax.random.key(42)
    k_rad, k_loop, k_acc = jax.random.split(key, 3)
    radius = jax.random.randint(k_rad, (bsize, 1), 1, 2 * _R)
    u_acc = jax.random.uniform(k_acc, (bsize,), dtype=jnp.float32)
    step_keys = jnp.stack(
        [jax.random.key_data(jax.random.fold_in(k_loop, t)) for t in range(_MAXR)])

    rblk = 128
    nblk = bsize // rblk

    rad_flat = radius[:, 0]
    perm = jnp.argsort(rad_flat)
    half = nblk // 2
    order = np.empty((nblk,), np.int32)
    order[0::2] = np.arange(half)
    order[1::2] = np.arange(nblk - 1, half - 1, -1)
    perm = perm.reshape(nblk, rblk)[order].reshape(-1)
    inv = jnp.argsort(perm)

    xp = x[perm]
    radp = rad_flat[perm][:, None]
    up = u_acc[perm][:, None]
    rowp = perm.astype(jnp.uint32)[:, None]

    body = functools.partial(_sampler_block, rblk=rblk, dim=dim)
    out_p = pl.pallas_call(
        body,
        grid=(nblk,),
        in_specs=[
            pl.BlockSpec((rblk, dim), lambda i: (i, 0)),
            pl.BlockSpec((1, dim), lambda i: (0, 0)),
            pl.BlockSpec((rblk, 1), lambda i: (i, 0)),
            pl.BlockSpec((rblk, 1), lambda i: (i, 0)),
            pl.BlockSpec((rblk, 1), lambda i: (i, 0)),
            pl.BlockSpec(memory_space=pltpu.SMEM),
        ],
        out_specs=pl.BlockSpec((rblk, dim), lambda i: (i, 0)),
        out_shape=jax.ShapeDtypeStruct((bsize, dim), jnp.float32),
        scratch_shapes=[pltpu.VMEM((rblk, dim), jnp.float32)],
        compiler_params=pltpu.CompilerParams(
            dimension_semantics=("parallel",),
        ),
    )(xp, W.reshape(1, dim), radp, up, rowp, step_keys)
    return out_p[inv]


# speedup vs baseline: 1.0463x; 1.0463x over previous
import functools

import jax
import jax.numpy as jnp
import numpy as np
from jax.experimental import pallas as pl
from jax.experimental.pallas import tpu as pltpu

_R = 10
_MAXR = 2 * _R - 1
_TINY = np.float32(np.finfo(np.float32).tiny)


def _threefry_xor_bits(k0, k1, cnt):
    ks2 = k0 ^ k1 ^ np.uint32(0x1BD11BDA)
    ks = (k0, k1, ks2)

    def rotl(v, d):
        return (v << np.uint32(d)) | (v >> np.uint32(32 - d))

    rots = ((13, 15, 26, 6), (17, 29, 16, 24))
    x1 = cnt + k1
    x0 = x1 + k0
    x1 = rotl(x1, 13)
    x1 = x1 ^ x0
    first = True
    for i in range(5):
        for r in rots[i % 2]:
            if first:
                first = False
                continue
            x0 = x0 + x1
            x1 = rotl(x1, r)
            x1 = x1 ^ x0
        x0 = x0 + ks[(i + 1) % 3]
        x1 = x1 + ks[(i + 2) % 3] + np.uint32(i + 1)
    return x0 ^ x1


def _gumbel_from_bits(bits):
    f = jax.lax.bitcast_convert_type(
        (bits >> np.uint32(9)) | np.uint32(0x3F800000), jnp.float32) - 1.0
    u = jnp.maximum(f, _TINY)
    return -jnp.log(-jnp.log(u))


def _sampler_block(x_ref, w_ref, rad_ref, u_ref, row_ref, keys_ref, o_ref, *, rblk, dim):
    x0 = x_ref[...]
    w = w_ref[...]
    wh = w * np.float32(0.5)

    def signed_logits(xb):
        return wh - xb * w

    col = jax.lax.broadcasted_iota(jnp.int32, (rblk, dim), 1)
    flat = row_ref[...] * np.uint32(dim) + \
        jax.lax.broadcasted_iota(jnp.uint32, (rblk, dim), 1)

    s0 = signed_logits(x0)
    m0 = jnp.max(s0, axis=-1, keepdims=True)
    log_zx = jnp.log(jnp.sum(jnp.exp(s0 - m0), axis=-1, keepdims=True)) + m0
    score_x = jnp.sum(x0 * w, axis=-1, keepdims=True)
    rad = rad_ref[...]
    t_max = jnp.max(rad)

    o_ref[...] = x0

    def step(t, carry):
        xc = o_ref[...]
        s = signed_logits(xc)
        bits = _threefry_xor_bits(keys_ref[t, 0], keys_ref[t, 1], flat)
        v = _gumbel_from_bits(bits) + s
        m = jnp.max(v, axis=-1, keepdims=True)
        idx = jnp.min(jnp.where(v == m, col, np.int32(dim)), axis=-1, keepdims=True)
        idx = jnp.where(t < rad, idx, np.int32(dim))
        mask = col == idx
        o_ref[...] = jnp.where(mask, 1.0 - xc, xc)
        return carry

    jax.lax.fori_loop(0, t_max, step, 0, unroll=False)

    y = o_ref[...]
    s_y = signed_logits(y)
    my = jnp.max(s_y, axis=-1, keepdims=True)
    lse_y = jnp.log(jnp.sum(jnp.exp(s_y - my), axis=-1, keepdims=True)) + my
    score_y = jnp.sum(y * w, axis=-1, keepdims=True)
    log_tilde = -jnp.sum(w * (y - x0), axis=-1, keepdims=True)
    log_acc = jnp.minimum((score_y - score_x) + log_tilde + (log_zx - lse_y), 0.0)
    acc = jnp.exp(log_acc) >= u_ref[...]
    o_ref[...] = jnp.where(acc, y, x0)


@jax.jit
def kernel(x, W):
    bsize, dim = x.shape
    key = jax.random.key(42)
    k_rad, k_loop, k_acc = jax.random.split(key, 3)
    radius = jax.random.randint(k_rad, (bsize, 1), 1, 2 * _R)
    u_acc = jax.random.uniform(k_acc, (bsize,), dtype=jnp.float32)
    step_keys = jnp.stack(
        [jax.random.key_data(jax.random.fold_in(k_loop, t)) for t in range(_MAXR)])

    rblk = 128
    nblk = bsize // rblk

    rad_flat = radius[:, 0]
    perm = jnp.argsort(rad_flat)
    half = nblk // 2
    order = np.empty((nblk,), np.int32)
    order[0::2] = np.arange(half)
    order[1::2] = np.arange(nblk - 1, half - 1, -1)
    perm = perm.reshape(nblk, rblk)[order].reshape(-1)
    inv = jnp.argsort(perm)

    xp = x[perm]
    radp = rad_flat[perm][:, None]
    up = u_acc[perm][:, None]
    rowp = perm.astype(jnp.uint32)[:, None]

    body = functools.partial(_sampler_block, rblk=rblk, dim=dim)
    out_p = pl.pallas_call(
        body,
        grid=(nblk,),
        in_specs=[
            pl.BlockSpec((rblk, dim), lambda i: (i, 0)),
            pl.BlockSpec((1, dim), lambda i: (0, 0)),
            pl.BlockSpec((rblk, 1), lambda i: (i, 0)),
            pl.BlockSpec((rblk, 1), lambda i: (i, 0)),
            pl.BlockSpec((rblk, 1), lambda i: (i, 0)),
            pl.BlockSpec(memory_space=pltpu.SMEM),
        ],
        out_specs=pl.BlockSpec((rblk, dim), lambda i: (i, 0)),
        out_shape=jax.ShapeDtypeStruct((bsize, dim), jnp.float32),
        compiler_params=pltpu.CompilerParams(
            dimension_semantics=("parallel",),
        ),
    )(xp, W.reshape(1, dim), radp, up, rowp, step_keys)
    return out_p[inv]
